# baseline (speedup 1.0000x reference)
import functools

import jax
import jax.numpy as jnp
from jax import lax
from jax.experimental import pallas as pl
from jax.experimental.pallas import tpu as pltpu
from jax.experimental.pallas import tpu_sc as plsc

VOCAB = 1000000
BATCH = 16384
SEQ = 50
EMBED_DIM = 64
PAD_DIM = 128

NUM_CORES = 2
NUM_SUBCORES = 16
NUM_WORKERS = NUM_CORES * NUM_SUBCORES

ROWS_PER_W = BATCH // NUM_WORKERS
NBUF = 2
LANES = 16


def _make_gather():
    mesh = plsc.VectorSubcoreMesh(
        core_axis_name="c", subcore_axis_name="s",
        num_cores=NUM_CORES, num_subcores=NUM_SUBCORES,
    )

    def body(idx_hbm, tdup_hbm, out_hbm, idx_v, gbuf, sbuf, *sems):
        gsems = sems[:NBUF]
        ssems = sems[NBUF:]
        wid = lax.axis_index("s") * NUM_CORES + lax.axis_index("c")
        base = wid * ROWS_PER_W

        pltpu.sync_copy(idx_hbm.at[pl.ds(base, ROWS_PER_W)], idx_v)

        for b in range(NBUF):
            pltpu.async_copy(tdup_hbm.at[idx_v.at[b]], gbuf.at[b], gsems[b])

        @pl.loop(0, ROWS_PER_W, step=NBUF)
        def _group(g):
            for b in range(NBUF):
                pltpu.make_async_copy(
                    tdup_hbm.at[idx_v.at[b]], gbuf.at[b], gsems[b]
                ).wait()

                @pl.loop(0, SEQ)
                def _compact(s):
                    for c in range(EMBED_DIM // LANES):
                        sbuf[b, s, pl.ds(c * LANES, LANES)] = gbuf[
                            b, s, pl.ds(c * LANES, LANES)
                        ]

                pltpu.async_copy(sbuf.at[b], out_hbm.at[base + g + b], ssems[b])
            for b in range(NBUF):
                pltpu.make_async_copy(
                    sbuf.at[b], out_hbm.at[0], ssems[b]
                ).wait()

                @pl.when(g + NBUF + b < ROWS_PER_W)
                def _refill():
                    pltpu.async_copy(
                        tdup_hbm.at[idx_v.at[g + NBUF + b]],
                        gbuf.at[b],
                        gsems[b],
                    )

    return pl.kernel(
        body,
        out_type=jax.ShapeDtypeStruct((BATCH, SEQ, EMBED_DIM), jnp.float32),
        mesh=mesh,
        scratch_types=[
            pltpu.VMEM((ROWS_PER_W, SEQ), jnp.int32),
            pltpu.VMEM((NBUF, SEQ, PAD_DIM), jnp.float32),
            pltpu.VMEM((NBUF, SEQ, EMBED_DIM), jnp.float32),
        ]
        + [pltpu.SemaphoreType.DMA] * (2 * NBUF),
    )


@jax.jit
def _lookup(batch_input, table):
    tdup = jnp.pad(table, ((0, 0), (0, PAD_DIM - EMBED_DIM)))
    return _make_gather()(batch_input, tdup)


def kernel(lang, batch_input, table):
    del lang
    return _lookup(batch_input, table)


# speedup vs baseline: 1.1224x; 1.1224x over previous
import functools

import jax
import jax.numpy as jnp
from jax import lax
from jax.experimental import pallas as pl
from jax.experimental.pallas import tpu as pltpu
from jax.experimental.pallas import tpu_sc as plsc

VOCAB = 1000000
BATCH = 16384
SEQ = 50
EMBED_DIM = 64
PAD_DIM = 128

NUM_CORES = 2
NUM_SUBCORES = 16
NUM_WORKERS = NUM_CORES * NUM_SUBCORES

ROWS_PER_W = BATCH // NUM_WORKERS
NBUF = 4
LANES = 16


def _make_gather():
    mesh = plsc.VectorSubcoreMesh(
        core_axis_name="c", subcore_axis_name="s",
        num_cores=NUM_CORES, num_subcores=NUM_SUBCORES,
    )

    def body(idx_hbm, tdup_hbm, out_hbm, idx_v, gbuf, sbuf, *sems):
        gsems = sems[:NBUF]
        ssems = sems[NBUF:]
        wid = lax.axis_index("s") * NUM_CORES + lax.axis_index("c")
        base = wid * ROWS_PER_W

        pltpu.sync_copy(idx_hbm.at[pl.ds(base, ROWS_PER_W)], idx_v)

        for b in range(NBUF):
            pltpu.async_copy(tdup_hbm.at[idx_v.at[b]], gbuf.at[b], gsems[b])

        @pl.loop(0, ROWS_PER_W, step=NBUF)
        def _group(g):
            for b in range(NBUF):
                pltpu.make_async_copy(
                    tdup_hbm.at[idx_v.at[b]], gbuf.at[b], gsems[b]
                ).wait()

                @pl.loop(0, SEQ)
                def _compact(s):
                    for c in range(EMBED_DIM // LANES):
                        sbuf[b, s, pl.ds(c * LANES, LANES)] = gbuf[
                            b, s, pl.ds(c * LANES, LANES)
                        ]

                pltpu.async_copy(sbuf.at[b], out_hbm.at[base + g + b], ssems[b])
            for b in range(NBUF):
                pltpu.make_async_copy(
                    sbuf.at[b], out_hbm.at[0], ssems[b]
                ).wait()

                @pl.when(g + NBUF + b < ROWS_PER_W)
                def _refill():
                    pltpu.async_copy(
                        tdup_hbm.at[idx_v.at[g + NBUF + b]],
                        gbuf.at[b],
                        gsems[b],
                    )

    return pl.kernel(
        body,
        out_type=jax.ShapeDtypeStruct((BATCH, SEQ, EMBED_DIM), jnp.float32),
        mesh=mesh,
        scratch_types=[
            pltpu.VMEM((ROWS_PER_W, SEQ), jnp.int32),
            pltpu.VMEM((NBUF, SEQ, PAD_DIM), jnp.float32),
            pltpu.VMEM((NBUF, SEQ, EMBED_DIM), jnp.float32),
        ]
        + [pltpu.SemaphoreType.DMA] * (2 * NBUF),
    )


@jax.jit
def _lookup(batch_input, table):
    tdup = jnp.pad(table, ((0, 0), (0, PAD_DIM - EMBED_DIM)))
    return _make_gather()(batch_input, tdup)


def kernel(lang, batch_input, table):
    del lang
    return _lookup(batch_input, table)
